# Initial kernel scaffold; baseline (speedup 1.0000x reference)
import jax
import jax.numpy as jnp
from jax.experimental import pallas as pl


def kernel(z_flat, codebook, embedding_avg, cluster_size):
    raise NotImplementedError("write your pallas kernel here")


# speedup vs baseline: 1.5033x; 1.5033x over previous
import functools

import jax
import jax.numpy as jnp
from jax import lax
from jax.experimental import pallas as pl
from jax.experimental.pallas import tpu as pltpu, tpu_sc as plsc

N = 8192
K = 8192
D = 256
DH = D // 2
DECAY = 0.99
EPS = 1e-05

BN = 1024
BK = 2048
NBLK = N // BN
KBLK = K // BK

NC = 2
NS = 16
CH = 128
TOK_W = N // (NC * NS)
TOK_S = N // NS
ROWS_S = K // NS


def _argmin_body(z_ref, cb_ref, out_ref):
    z = z_ref[...]
    z2 = jnp.sum(z * z, axis=1, keepdims=True)

    def body(j, carry):
        run_min, run_arg = carry
        c_blk = cb_ref[pl.ds(j * BK, BK), :]
        c2 = jnp.sum(c_blk * c_blk, axis=1)[None, :]
        zc = lax.dot_general(z, c_blk, (((1,), (1,)), ((), ())),
                             preferred_element_type=jnp.float32)
        dist = z2 + c2 - 2.0 * zc
        bmin = jnp.min(dist, axis=1, keepdims=True)
        col = lax.broadcasted_iota(jnp.int32, (BN, BK), 1) + j * BK
        cand = jnp.where(dist == bmin, col, K)
        barg = jnp.min(cand, axis=1, keepdims=True)
        upd = bmin < run_min
        return (jnp.where(upd, bmin, run_min), jnp.where(upd, barg, run_arg))

    init = (jnp.full((BN, 1), jnp.inf, jnp.float32),
            jnp.full((BN, 1), K, jnp.int32))
    _, run_arg = lax.fori_loop(0, KBLK, body, init)
    out_ref[0] = run_arg


def _argmin_call(z_flat, codebook):
    return pl.pallas_call(
        _argmin_body,
        grid=(NBLK,),
        in_specs=[
            pl.BlockSpec((BN, D), lambda i: (i, 0)),
            pl.BlockSpec((K, D), lambda i: (0, 0)),
        ],
        out_specs=pl.BlockSpec((1, BN, 1), lambda i: (i, 0, 0)),
        out_shape=jax.ShapeDtypeStruct((NBLK, BN, 1), jnp.int32),
        compiler_params=pltpu.CompilerParams(
            dimension_semantics=("arbitrary",)),
    )(z_flat, codebook)


def _sc_body(idx_hbm, z_hbm, cb_hbm, zrows_hbm, zcnt_hbm,
             quant_hbm, esum_hbm, counts_hbm,
             idx_v, rows_v, z_v, ones_v, table_sh, counts_sh, sem):
    c = lax.axis_index("c")
    s = lax.axis_index("s")
    w = c * NS + s

    for t in range(TOK_W // CH):
        base = w * TOK_W + t * CH
        pltpu.sync_copy(idx_hbm.at[pl.ds(base, CH)], idx_v)
        pltpu.async_copy(cb_hbm.at[idx_v], rows_v, sem).wait()
        pltpu.sync_copy(rows_v, quant_hbm.at[pl.ds(base, CH)])

    pltpu.sync_copy(zrows_hbm.at[pl.ds(s * ROWS_S, ROWS_S)],
                    table_sh.at[pl.ds(s * ROWS_S, ROWS_S)])
    pltpu.sync_copy(zcnt_hbm.at[pl.ds(s * ROWS_S, ROWS_S)],
                    counts_sh.at[pl.ds(s * ROWS_S, ROWS_S)])
    for j in range(CH // 16):
        ones_v[pl.ds(j * 16, 16)] = jnp.ones((16,), jnp.float32)
    plsc.subcore_barrier()

    for t in range(TOK_S // CH):
        base = s * TOK_S + t * CH
        pltpu.sync_copy(idx_hbm.at[pl.ds(base, CH)], idx_v)
        pltpu.sync_copy(z_hbm.at[pl.ds(base, CH), pl.ds(c * DH, DH)], z_v)
        pltpu.sync_copy(z_v, table_sh.at[idx_v], add=True)
        pltpu.sync_copy(ones_v, counts_sh.at[idx_v], add=True)
    plsc.subcore_barrier()

    pltpu.sync_copy(table_sh.at[pl.ds(s * ROWS_S, ROWS_S)],
                    esum_hbm.at[pl.ds(s * ROWS_S, ROWS_S), pl.ds(c * DH, DH)])

    @pl.when(c == 0)
    def _():
        pltpu.sync_copy(counts_sh.at[pl.ds(s * ROWS_S, ROWS_S)],
                        counts_hbm.at[pl.ds(s * ROWS_S, ROWS_S)])


def _sc_call(indices, z_flat, codebook, zrows, zcnt):
    mesh = plsc.VectorSubcoreMesh(core_axis_name="c", subcore_axis_name="s")
    f = functools.partial(
        pl.kernel,
        out_type=[
            jax.ShapeDtypeStruct((N, D), jnp.float32),
            jax.ShapeDtypeStruct((K, D), jnp.float32),
            jax.ShapeDtypeStruct((K,), jnp.float32),
        ],
        mesh=mesh,
        scratch_types=[
            pltpu.VMEM((CH,), jnp.int32),
            pltpu.VMEM((CH, D), jnp.float32),
            pltpu.VMEM((CH, DH), jnp.float32),
            pltpu.VMEM((CH,), jnp.float32),
            pltpu.VMEM_SHARED((K, DH), jnp.float32),
            pltpu.VMEM_SHARED((K,), jnp.float32),
            pltpu.SemaphoreType.DMA,
        ],
    )(_sc_body)
    return f(indices, z_flat, codebook, zrows, zcnt)


def _combine_body(cnt_ref, cls_ref, esum_ref, eavg_ref,
                  csn_ref, avg_ref, cb_ref):
    i = pl.program_id(0)
    csn_full = cls_ref[...] * DECAY + cnt_ref[...] * (1.0 - DECAY)
    n = jnp.sum(csn_full)
    csn_blk = (cls_ref[pl.ds(i * BN, BN), :] * DECAY
               + cnt_ref[pl.ds(i * BN, BN), :] * (1.0 - DECAY))
    csn_ref[...] = csn_blk
    cs = (csn_blk + EPS) / (n + K * EPS) * n
    avg_new = eavg_ref[...] * DECAY + esum_ref[...] * (1.0 - DECAY)
    avg_ref[...] = avg_new
    cb_ref[...] = avg_new / cs


def _combine_call(counts, cluster_size, embed_sum, embedding_avg):
    return pl.pallas_call(
        _combine_body,
        grid=(NBLK,),
        in_specs=[
            pl.BlockSpec((K, 1), lambda i: (0, 0)),
            pl.BlockSpec((K, 1), lambda i: (0, 0)),
            pl.BlockSpec((BN, D), lambda i: (i, 0)),
            pl.BlockSpec((BN, D), lambda i: (i, 0)),
        ],
        out_specs=[
            pl.BlockSpec((BN, 1), lambda i: (i, 0)),
            pl.BlockSpec((BN, D), lambda i: (i, 0)),
            pl.BlockSpec((BN, D), lambda i: (i, 0)),
        ],
        out_shape=[
            jax.ShapeDtypeStruct((K, 1), jnp.float32),
            jax.ShapeDtypeStruct((K, D), jnp.float32),
            jax.ShapeDtypeStruct((K, D), jnp.float32),
        ],
        compiler_params=pltpu.CompilerParams(
            dimension_semantics=("arbitrary",)),
    )(counts, cluster_size, embed_sum, embedding_avg)


def kernel(z_flat, codebook, embedding_avg, cluster_size):
    indices = _argmin_call(z_flat, codebook).reshape(N)
    zrows = jnp.zeros((K, DH), jnp.float32)
    zcnt = jnp.zeros((K,), jnp.float32)
    quantized, embed_sum, counts = _sc_call(indices, z_flat, codebook,
                                            zrows, zcnt)
    csn, avg_new, cb_new = _combine_call(counts.reshape(K, 1),
                                         cluster_size.reshape(K, 1),
                                         embed_sum, embedding_avg)
    return quantized, indices, cb_new, csn.reshape(K), avg_new


# speedup vs baseline: 1.7086x; 1.1366x over previous
import functools

import jax
import jax.numpy as jnp
from jax import lax
from jax.experimental import pallas as pl
from jax.experimental.pallas import tpu as pltpu, tpu_sc as plsc

N = 8192
K = 8192
D = 256
DH = D // 2
DECAY = 0.99
EPS = 1e-05

BN = 1024
BK = 2048
NBLK = N // BN
KBLK = K // BK

NC = 2
NS = 16
CH = 128
TOK_W = N // (NC * NS)
TOK_S = N // NS
ROWS_S = K // NS


def _argmin_body(z_ref, cb_ref, out_ref):
    z = z_ref[...]
    z2 = jnp.sum(z * z, axis=1, keepdims=True)
    z2x = 2.0 * z
    lane = lax.broadcasted_iota(jnp.int32, (1, 128), 1)

    def body(j, carry):
        run_val, run_idx = carry
        c_blk = cb_ref[pl.ds(j * BK, BK), :]
        c2 = jnp.sum(c_blk * c_blk, axis=1)[None, :]
        zc2 = lax.dot_general(z2x, c_blk, (((1,), (1,)), ((), ())),
                              preferred_element_type=jnp.float32)
        dist = (z2 + c2) - zc2
        for g in range(BK // 128):
            dg = dist[:, g * 128:(g + 1) * 128]
            m = dg < run_val
            run_val = jnp.where(m, dg, run_val)
            run_idx = jnp.where(m, lane + (j * BK + g * 128), run_idx)
        return run_val, run_idx

    init = (jnp.full((BN, 128), jnp.inf, jnp.float32),
            jnp.full((BN, 128), K, jnp.int32))
    run_val, run_idx = lax.fori_loop(0, KBLK, body, init)
    bmin = jnp.min(run_val, axis=1, keepdims=True)
    cand = jnp.where(run_val == bmin, run_idx, K)
    out_ref[0] = jnp.min(cand, axis=1, keepdims=True)


def _argmin_call(z_flat, codebook):
    return pl.pallas_call(
        _argmin_body,
        grid=(NBLK,),
        in_specs=[
            pl.BlockSpec((BN, D), lambda i: (i, 0)),
            pl.BlockSpec((K, D), lambda i: (0, 0)),
        ],
        out_specs=pl.BlockSpec((1, BN, 1), lambda i: (i, 0, 0)),
        out_shape=jax.ShapeDtypeStruct((NBLK, BN, 1), jnp.int32),
        compiler_params=pltpu.CompilerParams(
            dimension_semantics=("arbitrary",)),
    )(z_flat, codebook)


def _sc_body(idx_hbm, z_hbm, cb_hbm, zrows_hbm, zcnt_hbm,
             quant_hbm, esum_hbm, counts_hbm,
             idx_v, rows_v, z_v, ones_v, table_sh, counts_sh, sem):
    c = lax.axis_index("c")
    s = lax.axis_index("s")
    w = c * NS + s

    for t in range(TOK_W // CH):
        base = w * TOK_W + t * CH
        pltpu.sync_copy(idx_hbm.at[pl.ds(base, CH)], idx_v)
        pltpu.async_copy(cb_hbm.at[idx_v], rows_v, sem).wait()
        pltpu.sync_copy(rows_v, quant_hbm.at[pl.ds(base, CH)])

    pltpu.sync_copy(zrows_hbm.at[pl.ds(s * ROWS_S, ROWS_S)],
                    table_sh.at[pl.ds(s * ROWS_S, ROWS_S)])
    pltpu.sync_copy(zcnt_hbm.at[pl.ds(s * ROWS_S, ROWS_S)],
                    counts_sh.at[pl.ds(s * ROWS_S, ROWS_S)])
    for j in range(CH // 16):
        ones_v[pl.ds(j * 16, 16)] = jnp.ones((16,), jnp.float32)
    plsc.subcore_barrier()

    for t in range(TOK_S // CH):
        base = s * TOK_S + t * CH
        pltpu.sync_copy(idx_hbm.at[pl.ds(base, CH)], idx_v)
        pltpu.sync_copy(z_hbm.at[pl.ds(base, CH), pl.ds(c * DH, DH)], z_v)
        pltpu.sync_copy(z_v, table_sh.at[idx_v], add=True)
        pltpu.sync_copy(ones_v, counts_sh.at[idx_v], add=True)
    plsc.subcore_barrier()

    pltpu.sync_copy(table_sh.at[pl.ds(s * ROWS_S, ROWS_S)],
                    esum_hbm.at[pl.ds(s * ROWS_S, ROWS_S), pl.ds(c * DH, DH)])

    @pl.when(c == 0)
    def _():
        pltpu.sync_copy(counts_sh.at[pl.ds(s * ROWS_S, ROWS_S)],
                        counts_hbm.at[pl.ds(s * ROWS_S, ROWS_S)])


def _sc_call(indices, z_flat, codebook, zrows, zcnt):
    mesh = plsc.VectorSubcoreMesh(core_axis_name="c", subcore_axis_name="s")
    f = functools.partial(
        pl.kernel,
        out_type=[
            jax.ShapeDtypeStruct((N, D), jnp.float32),
            jax.ShapeDtypeStruct((K, D), jnp.float32),
            jax.ShapeDtypeStruct((K,), jnp.float32),
        ],
        mesh=mesh,
        scratch_types=[
            pltpu.VMEM((CH,), jnp.int32),
            pltpu.VMEM((CH, D), jnp.float32),
            pltpu.VMEM((CH, DH), jnp.float32),
            pltpu.VMEM((CH,), jnp.float32),
            pltpu.VMEM_SHARED((K, DH), jnp.float32),
            pltpu.VMEM_SHARED((K,), jnp.float32),
            pltpu.SemaphoreType.DMA,
        ],
    )(_sc_body)
    return f(indices, z_flat, codebook, zrows, zcnt)


def _combine_body(cnt_ref, cls_ref, esum_ref, eavg_ref,
                  csn_ref, avg_ref, cb_ref):
    i = pl.program_id(0)
    csn_full = cls_ref[...] * DECAY + cnt_ref[...] * (1.0 - DECAY)
    n = jnp.sum(csn_full)
    csn_blk = (cls_ref[pl.ds(i * BN, BN), :] * DECAY
               + cnt_ref[pl.ds(i * BN, BN), :] * (1.0 - DECAY))
    csn_ref[...] = csn_blk
    cs = (csn_blk + EPS) / (n + K * EPS) * n
    avg_new = eavg_ref[...] * DECAY + esum_ref[...] * (1.0 - DECAY)
    avg_ref[...] = avg_new
    cb_ref[...] = avg_new / cs


def _combine_call(counts, cluster_size, embed_sum, embedding_avg):
    return pl.pallas_call(
        _combine_body,
        grid=(NBLK,),
        in_specs=[
            pl.BlockSpec((K, 1), lambda i: (0, 0)),
            pl.BlockSpec((K, 1), lambda i: (0, 0)),
            pl.BlockSpec((BN, D), lambda i: (i, 0)),
            pl.BlockSpec((BN, D), lambda i: (i, 0)),
        ],
        out_specs=[
            pl.BlockSpec((BN, 1), lambda i: (i, 0)),
            pl.BlockSpec((BN, D), lambda i: (i, 0)),
            pl.BlockSpec((BN, D), lambda i: (i, 0)),
        ],
        out_shape=[
            jax.ShapeDtypeStruct((K, 1), jnp.float32),
            jax.ShapeDtypeStruct((K, D), jnp.float32),
            jax.ShapeDtypeStruct((K, D), jnp.float32),
        ],
        compiler_params=pltpu.CompilerParams(
            dimension_semantics=("arbitrary",)),
    )(counts, cluster_size, embed_sum, embedding_avg)


def kernel(z_flat, codebook, embedding_avg, cluster_size):
    indices = _argmin_call(z_flat, codebook).reshape(N)
    zrows = jnp.zeros((K, DH), jnp.float32)
    zcnt = jnp.zeros((K,), jnp.float32)
    quantized, embed_sum, counts = _sc_call(indices, z_flat, codebook,
                                            zrows, zcnt)
    csn, avg_new, cb_new = _combine_call(counts.reshape(K, 1),
                                         cluster_size.reshape(K, 1),
                                         embed_sum, embedding_avg)
    return quantized, indices, cb_new, csn.reshape(K), avg_new


# speedup vs baseline: 1.8266x; 1.0691x over previous
import functools

import jax
import jax.numpy as jnp
from jax import lax
from jax.experimental import pallas as pl
from jax.experimental.pallas import tpu as pltpu, tpu_sc as plsc

N = 8192
K = 8192
D = 256
DH = D // 2
DECAY = 0.99
EPS = 1e-05

BN = 1024
BK = 2048
NBLK = N // BN
KBLK = K // BK

NC = 2
NS = 16
CH = 128
TOK_W = N // (NC * NS)
TOK_S = N // NS
ROWS_S = K // NS


def _argmin_body(z_ref, cb_ref, out_ref):
    z = z_ref[...]
    z2 = jnp.sum(z * z, axis=1, keepdims=True)
    z2x = 2.0 * z
    lane = lax.broadcasted_iota(jnp.int32, (1, 128), 1)

    def body(j, carry):
        run_val, run_idx = carry
        c_blk = cb_ref[pl.ds(j * BK, BK), :]
        c2 = jnp.sum(c_blk * c_blk, axis=1)[None, :]
        zc2 = lax.dot_general(z2x, c_blk, (((1,), (1,)), ((), ())),
                              preferred_element_type=jnp.float32)
        dist = (z2 + c2) - zc2
        for g in range(BK // 128):
            dg = dist[:, g * 128:(g + 1) * 128]
            m = dg < run_val
            run_val = jnp.where(m, dg, run_val)
            run_idx = jnp.where(m, lane + (j * BK + g * 128), run_idx)
        return run_val, run_idx

    init = (jnp.full((BN, 128), jnp.inf, jnp.float32),
            jnp.full((BN, 128), K, jnp.int32))
    run_val, run_idx = lax.fori_loop(0, KBLK, body, init)
    bmin = jnp.min(run_val, axis=1, keepdims=True)
    cand = jnp.where(run_val == bmin, run_idx, K)
    out_ref[0] = jnp.min(cand, axis=1, keepdims=True)


def _argmin_call(z_flat, codebook):
    return pl.pallas_call(
        _argmin_body,
        grid=(NBLK,),
        in_specs=[
            pl.BlockSpec((BN, D), lambda i: (i, 0)),
            pl.BlockSpec((K, D), lambda i: (0, 0)),
        ],
        out_specs=pl.BlockSpec((1, BN, 1), lambda i: (i, 0, 0)),
        out_shape=jax.ShapeDtypeStruct((NBLK, BN, 1), jnp.int32),
        compiler_params=pltpu.CompilerParams(
            dimension_semantics=("arbitrary",)),
    )(z_flat, codebook)


CHS = 64
CHG = 64


def _sc_body(idx_hbm, z_hbm, cb_hbm,
             quant_hbm, esum_hbm, counts_hbm,
             sidx_a, sidx_b, z_a, z_b, gidx_a, gidx_b, rows_a, rows_b,
             zbuf, zcnt_v, ones_v,
             table_sh, counts_sh,
             semz, sxi_a, sxi_b, sxz_a, sxz_b,
             sgi_a, sgi_b, sgg_a, sgg_b, sgs_a, sgs_b):
    c = lax.axis_index("c")
    s = lax.axis_index("s")
    w = c * NS + s

    for r in range(32):
        for j in range(DH // 16):
            zbuf[r, pl.ds(j * 16, 16)] = jnp.zeros((16,), jnp.float32)
    for j in range(ROWS_S // 16):
        zcnt_v[pl.ds(j * 16, 16)] = jnp.zeros((16,), jnp.float32)
    for j in range(CHS // 16):
        ones_v[pl.ds(j * 16, 16)] = jnp.ones((16,), jnp.float32)

    zd = []
    for q in range(ROWS_S // 32):
        zd.append(pltpu.async_copy(
            zbuf, table_sh.at[pl.ds(s * ROWS_S + q * 32, 32)], semz))
    zd.append(pltpu.async_copy(
        zcnt_v, counts_sh.at[pl.ds(s * ROWS_S, ROWS_S)], semz))

    nch = TOK_S // CHS
    sbase = s * TOK_S
    bufs = [(sidx_a, z_a, sxi_a, sxz_a), (sidx_b, z_b, sxi_b, sxz_b)]
    loads = {}

    def issue_load(t):
        ib, zb_, si, sz = bufs[t % 2]
        b = sbase + t * CHS
        loads[t] = (pltpu.async_copy(idx_hbm.at[pl.ds(b, CHS)], ib, si),
                    pltpu.async_copy(
                        z_hbm.at[pl.ds(b, CHS), pl.ds(c * DH, DH)], zb_, sz))

    issue_load(0)
    for d in zd:
        d.wait()
    plsc.subcore_barrier()

    for t in range(nch):
        if t + 1 < nch:
            issue_load(t + 1)
        di, dz = loads.pop(t)
        di.wait()
        dz.wait()
        ib, zb_, _, _ = bufs[t % 2]
        pltpu.sync_copy(zb_, table_sh.at[ib], add=True)
        pltpu.sync_copy(ones_v, counts_sh.at[ib], add=True)
    plsc.subcore_barrier()

    dr = pltpu.async_copy(
        table_sh.at[pl.ds(s * ROWS_S, ROWS_S)],
        esum_hbm.at[pl.ds(s * ROWS_S, ROWS_S), pl.ds(c * DH, DH)], semz)

    @pl.when(c == 0)
    def _():
        pltpu.sync_copy(counts_sh.at[pl.ds(s * ROWS_S, ROWS_S)],
                        counts_hbm.at[pl.ds(s * ROWS_S, ROWS_S)])

    ngc = TOK_W // CHG
    gbufs = [(gidx_a, rows_a, sgi_a, sgg_a, sgs_a),
             (gidx_b, rows_b, sgi_b, sgg_b, sgs_b)]
    il = {}
    gl = {}
    st = {}

    def g_issue_idx(t):
        gi, _, si, _, _ = gbufs[t % 2]
        il[t] = pltpu.async_copy(
            idx_hbm.at[pl.ds(w * TOK_W + t * CHG, CHG)], gi, si)

    g_issue_idx(0)
    for t in range(ngc):
        gi, rw, _, sg, ss = gbufs[t % 2]
        if t >= 2:
            st[t - 2].wait()
        il[t].wait()
        if t + 1 < ngc:
            g_issue_idx(t + 1)
        gl[t] = pltpu.async_copy(cb_hbm.at[gi], rw, sg)
        gl[t].wait()
        st[t] = pltpu.async_copy(
            rw, quant_hbm.at[pl.ds(w * TOK_W + t * CHG, CHG)], ss)
    for t in (ngc - 2, ngc - 1):
        st[t].wait()
    dr.wait()


def _sc_call(indices, z_flat, codebook):
    mesh = plsc.VectorSubcoreMesh(core_axis_name="c", subcore_axis_name="s")
    f = functools.partial(
        pl.kernel,
        out_type=[
            jax.ShapeDtypeStruct((N, D), jnp.float32),
            jax.ShapeDtypeStruct((K, D), jnp.float32),
            jax.ShapeDtypeStruct((K,), jnp.float32),
        ],
        mesh=mesh,
        scratch_types=[
            pltpu.VMEM((CHS,), jnp.int32),
            pltpu.VMEM((CHS,), jnp.int32),
            pltpu.VMEM((CHS, DH), jnp.float32),
            pltpu.VMEM((CHS, DH), jnp.float32),
            pltpu.VMEM((CHG,), jnp.int32),
            pltpu.VMEM((CHG,), jnp.int32),
            pltpu.VMEM((CHG, D), jnp.float32),
            pltpu.VMEM((CHG, D), jnp.float32),
            pltpu.VMEM((32, DH), jnp.float32),
            pltpu.VMEM((ROWS_S,), jnp.float32),
            pltpu.VMEM((CHS,), jnp.float32),
            pltpu.VMEM_SHARED((K, DH), jnp.float32),
            pltpu.VMEM_SHARED((K,), jnp.float32),
        ] + [pltpu.SemaphoreType.DMA] * 11,
    )(_sc_body)
    return f(indices, z_flat, codebook)


def _combine_body(cnt_ref, cls_ref, esum_ref, eavg_ref,
                  csn_ref, avg_ref, cb_ref):
    i = pl.program_id(0)
    csn_full = cls_ref[...] * DECAY + cnt_ref[...] * (1.0 - DECAY)
    n = jnp.sum(csn_full)
    csn_blk = (cls_ref[pl.ds(i * BN, BN), :] * DECAY
               + cnt_ref[pl.ds(i * BN, BN), :] * (1.0 - DECAY))
    csn_ref[...] = csn_blk
    cs = (csn_blk + EPS) / (n + K * EPS) * n
    avg_new = eavg_ref[...] * DECAY + esum_ref[...] * (1.0 - DECAY)
    avg_ref[...] = avg_new
    cb_ref[...] = avg_new / cs


def _combine_call(counts, cluster_size, embed_sum, embedding_avg):
    return pl.pallas_call(
        _combine_body,
        grid=(NBLK,),
        in_specs=[
            pl.BlockSpec((K, 1), lambda i: (0, 0)),
            pl.BlockSpec((K, 1), lambda i: (0, 0)),
            pl.BlockSpec((BN, D), lambda i: (i, 0)),
            pl.BlockSpec((BN, D), lambda i: (i, 0)),
        ],
        out_specs=[
            pl.BlockSpec((BN, 1), lambda i: (i, 0)),
            pl.BlockSpec((BN, D), lambda i: (i, 0)),
            pl.BlockSpec((BN, D), lambda i: (i, 0)),
        ],
        out_shape=[
            jax.ShapeDtypeStruct((K, 1), jnp.float32),
            jax.ShapeDtypeStruct((K, D), jnp.float32),
            jax.ShapeDtypeStruct((K, D), jnp.float32),
        ],
        compiler_params=pltpu.CompilerParams(
            dimension_semantics=("arbitrary",)),
    )(counts, cluster_size, embed_sum, embedding_avg)


def kernel(z_flat, codebook, embedding_avg, cluster_size):
    indices = _argmin_call(z_flat, codebook).reshape(N)
    quantized, embed_sum, counts = _sc_call(indices, z_flat, codebook)
    csn, avg_new, cb_new = _combine_call(counts.reshape(K, 1),
                                         cluster_size.reshape(K, 1),
                                         embed_sum, embedding_avg)
    return quantized, indices, cb_new, csn.reshape(K), avg_new


# speedup vs baseline: 1.8339x; 1.0040x over previous
import functools

import jax
import jax.numpy as jnp
from jax import lax
from jax.experimental import pallas as pl
from jax.experimental.pallas import tpu as pltpu, tpu_sc as plsc

N = 8192
K = 8192
D = 256
DH = D // 2
DECAY = 0.99
EPS = 1e-05

BN = 1024
BK = 2048
NBLK = N // BN
KBLK = K // BK

NC = 2
NS = 16
CH = 128
TOK_W = N // (NC * NS)
TOK_S = N // NS
ROWS_S = K // NS


def _argmin_body(z_ref, cb_ref, out_ref, c2_ref):
    i = pl.program_id(0)

    @pl.when(i == 0)
    def _():
        for j in range(KBLK):
            c_blk = cb_ref[pl.ds(j * BK, BK), :]
            c2_ref[:, pl.ds(j * BK, BK)] = (
                jnp.sum(c_blk * c_blk, axis=1)[None, :])

    z = z_ref[...]
    z2 = jnp.sum(z * z, axis=1, keepdims=True)
    z2x = 2.0 * z
    lane = lax.broadcasted_iota(jnp.int32, (1, 128), 1)

    def body(j, carry):
        run_val, run_gid = carry
        c_blk = cb_ref[pl.ds(j * BK, BK), :]
        zc2 = lax.dot_general(z2x, c_blk, (((1,), (1,)), ((), ())),
                              preferred_element_type=jnp.float32)
        dist = (z2 + c2_ref[:, pl.ds(j * BK, BK)]) - zc2
        for g in range(BK // 128):
            dg = dist[:, g * 128:(g + 1) * 128]
            m = dg < run_val
            run_val = jnp.where(m, dg, run_val)
            run_gid = jnp.where(m, jnp.int32(j * (BK // 128) + g), run_gid)
        return run_val, run_gid

    init = (jnp.full((BN, 128), jnp.inf, jnp.float32),
            jnp.full((BN, 128), K // 128, jnp.int32))
    run_val, run_gid = lax.fori_loop(0, KBLK, body, init)
    col = run_gid * 128 + lane
    bmin = jnp.min(run_val, axis=1, keepdims=True)
    cand = jnp.where(run_val == bmin, col, K)
    out_ref[0] = jnp.min(cand, axis=1, keepdims=True)


def _argmin_call(z_flat, codebook):
    return pl.pallas_call(
        _argmin_body,
        grid=(NBLK,),
        in_specs=[
            pl.BlockSpec((BN, D), lambda i: (i, 0)),
            pl.BlockSpec((K, D), lambda i: (0, 0)),
        ],
        out_specs=pl.BlockSpec((1, BN, 1), lambda i: (i, 0, 0)),
        out_shape=jax.ShapeDtypeStruct((NBLK, BN, 1), jnp.int32),
        scratch_shapes=[pltpu.VMEM((1, K), jnp.float32)],
        compiler_params=pltpu.CompilerParams(
            dimension_semantics=("arbitrary",)),
    )(z_flat, codebook)


CHS = 64
CHG = 64


def _sc_body(idx_hbm, z_hbm, cb_hbm,
             quant_hbm, esum_hbm, counts_hbm,
             sidx_a, sidx_b, z_a, z_b, gidx_a, gidx_b, rows_a, rows_b,
             zbuf, zcnt_v, ones_v,
             table_sh, counts_sh,
             semz, sxi_a, sxi_b, sxz_a, sxz_b,
             sgi_a, sgi_b, sgg_a, sgg_b, sgs_a, sgs_b):
    c = lax.axis_index("c")
    s = lax.axis_index("s")
    w = c * NS + s

    for r in range(32):
        for j in range(DH // 16):
            zbuf[r, pl.ds(j * 16, 16)] = jnp.zeros((16,), jnp.float32)
    for j in range(ROWS_S // 16):
        zcnt_v[pl.ds(j * 16, 16)] = jnp.zeros((16,), jnp.float32)
    for j in range(CHS // 16):
        ones_v[pl.ds(j * 16, 16)] = jnp.ones((16,), jnp.float32)

    zd = []
    for q in range(ROWS_S // 32):
        zd.append(pltpu.async_copy(
            zbuf, table_sh.at[pl.ds(s * ROWS_S + q * 32, 32)], semz))
    zd.append(pltpu.async_copy(
        zcnt_v, counts_sh.at[pl.ds(s * ROWS_S, ROWS_S)], semz))

    nch = TOK_S // CHS
    sbase = s * TOK_S
    bufs = [(sidx_a, z_a, sxi_a, sxz_a), (sidx_b, z_b, sxi_b, sxz_b)]
    loads = {}

    def issue_load(t):
        ib, zb_, si, sz = bufs[t % 2]
        b = sbase + t * CHS
        loads[t] = (pltpu.async_copy(idx_hbm.at[pl.ds(b, CHS)], ib, si),
                    pltpu.async_copy(
                        z_hbm.at[pl.ds(b, CHS), pl.ds(c * DH, DH)], zb_, sz))

    issue_load(0)
    for d in zd:
        d.wait()
    plsc.subcore_barrier()

    for t in range(nch):
        if t + 1 < nch:
            issue_load(t + 1)
        di, dz = loads.pop(t)
        di.wait()
        dz.wait()
        ib, zb_, _, _ = bufs[t % 2]
        pltpu.sync_copy(zb_, table_sh.at[ib], add=True)
        pltpu.sync_copy(ones_v, counts_sh.at[ib], add=True)
    plsc.subcore_barrier()

    dr = pltpu.async_copy(
        table_sh.at[pl.ds(s * ROWS_S, ROWS_S)],
        esum_hbm.at[pl.ds(s * ROWS_S, ROWS_S), pl.ds(c * DH, DH)], semz)

    @pl.when(c == 0)
    def _():
        pltpu.sync_copy(counts_sh.at[pl.ds(s * ROWS_S, ROWS_S)],
                        counts_hbm.at[pl.ds(s * ROWS_S, ROWS_S)])

    ngc = TOK_W // CHG
    gbufs = [(gidx_a, rows_a, sgi_a, sgg_a, sgs_a),
             (gidx_b, rows_b, sgi_b, sgg_b, sgs_b)]
    il = {}
    gl = {}
    st = {}

    def g_issue_idx(t):
        gi, _, si, _, _ = gbufs[t % 2]
        il[t] = pltpu.async_copy(
            idx_hbm.at[pl.ds(w * TOK_W + t * CHG, CHG)], gi, si)

    g_issue_idx(0)
    for t in range(ngc):
        gi, rw, _, sg, ss = gbufs[t % 2]
        if t >= 2:
            st[t - 2].wait()
        il[t].wait()
        if t + 1 < ngc:
            g_issue_idx(t + 1)
        gl[t] = pltpu.async_copy(cb_hbm.at[gi], rw, sg)
        gl[t].wait()
        st[t] = pltpu.async_copy(
            rw, quant_hbm.at[pl.ds(w * TOK_W + t * CHG, CHG)], ss)
    for t in (ngc - 2, ngc - 1):
        st[t].wait()
    dr.wait()


def _sc_call(indices, z_flat, codebook):
    mesh = plsc.VectorSubcoreMesh(core_axis_name="c", subcore_axis_name="s")
    f = functools.partial(
        pl.kernel,
        out_type=[
            jax.ShapeDtypeStruct((N, D), jnp.float32),
            jax.ShapeDtypeStruct((K, D), jnp.float32),
            jax.ShapeDtypeStruct((K,), jnp.float32),
        ],
        mesh=mesh,
        scratch_types=[
            pltpu.VMEM((CHS,), jnp.int32),
            pltpu.VMEM((CHS,), jnp.int32),
            pltpu.VMEM((CHS, DH), jnp.float32),
            pltpu.VMEM((CHS, DH), jnp.float32),
            pltpu.VMEM((CHG,), jnp.int32),
            pltpu.VMEM((CHG,), jnp.int32),
            pltpu.VMEM((CHG, D), jnp.float32),
            pltpu.VMEM((CHG, D), jnp.float32),
            pltpu.VMEM((32, DH), jnp.float32),
            pltpu.VMEM((ROWS_S,), jnp.float32),
            pltpu.VMEM((CHS,), jnp.float32),
            pltpu.VMEM_SHARED((K, DH), jnp.float32),
            pltpu.VMEM_SHARED((K,), jnp.float32),
        ] + [pltpu.SemaphoreType.DMA] * 11,
    )(_sc_body)
    return f(indices, z_flat, codebook)


def _combine_body(cnt_ref, cls_ref, esum_ref, eavg_ref,
                  csn_ref, avg_ref, cb_ref):
    i = pl.program_id(0)
    csn_full = cls_ref[...] * DECAY + cnt_ref[...] * (1.0 - DECAY)
    n = jnp.sum(csn_full)
    csn_blk = (cls_ref[pl.ds(i * BN, BN), :] * DECAY
               + cnt_ref[pl.ds(i * BN, BN), :] * (1.0 - DECAY))
    csn_ref[...] = csn_blk
    cs = (csn_blk + EPS) / (n + K * EPS) * n
    avg_new = eavg_ref[...] * DECAY + esum_ref[...] * (1.0 - DECAY)
    avg_ref[...] = avg_new
    cb_ref[...] = avg_new / cs


def _combine_call(counts, cluster_size, embed_sum, embedding_avg):
    return pl.pallas_call(
        _combine_body,
        grid=(NBLK,),
        in_specs=[
            pl.BlockSpec((K, 1), lambda i: (0, 0)),
            pl.BlockSpec((K, 1), lambda i: (0, 0)),
            pl.BlockSpec((BN, D), lambda i: (i, 0)),
            pl.BlockSpec((BN, D), lambda i: (i, 0)),
        ],
        out_specs=[
            pl.BlockSpec((BN, 1), lambda i: (i, 0)),
            pl.BlockSpec((BN, D), lambda i: (i, 0)),
            pl.BlockSpec((BN, D), lambda i: (i, 0)),
        ],
        out_shape=[
            jax.ShapeDtypeStruct((K, 1), jnp.float32),
            jax.ShapeDtypeStruct((K, D), jnp.float32),
            jax.ShapeDtypeStruct((K, D), jnp.float32),
        ],
        compiler_params=pltpu.CompilerParams(
            dimension_semantics=("arbitrary",)),
    )(counts, cluster_size, embed_sum, embedding_avg)


def kernel(z_flat, codebook, embedding_avg, cluster_size):
    indices = _argmin_call(z_flat, codebook).reshape(N)
    quantized, embed_sum, counts = _sc_call(indices, z_flat, codebook)
    csn, avg_new, cb_new = _combine_call(counts.reshape(K, 1),
                                         cluster_size.reshape(K, 1),
                                         embed_sum, embedding_avg)
    return quantized, indices, cb_new, csn.reshape(K), avg_new
